# baseline (speedup 1.0000x reference)
import jax
import jax.numpy as jnp
from jax.experimental import pallas as pl

E = 8
K = 2
BLK = 2048


def _gate_block(hid_ref, w_ref, idx_ref, wgt_ref):
    h = hid_ref[...]
    w = w_ref[...]
    logits = jax.lax.dot_general(
        h, w, (((1,), (1,)), ((), ())),
        preferred_element_type=jnp.float32)
    m = jnp.max(logits, axis=-1, keepdims=True)
    s = jnp.exp(logits - m)
    z = jnp.sum(s, axis=-1, keepdims=True)
    scores = s / z

    lane = jax.lax.broadcasted_iota(jnp.int32, scores.shape, 1)
    w1 = jnp.max(scores, axis=-1, keepdims=True)
    i1 = jnp.min(jnp.where(scores >= w1, lane, E), axis=-1, keepdims=True)
    masked = jnp.where(lane == i1, -jnp.inf, scores)
    w2 = jnp.max(masked, axis=-1, keepdims=True)
    i2 = jnp.min(jnp.where(masked >= w2, lane, E), axis=-1, keepdims=True)

    denom = w1 + w2 + 1e-20
    idx_ref[...] = jnp.concatenate([i1, i2], axis=-1)
    wgt_ref[...] = jnp.concatenate([w1 / denom, w2 / denom], axis=-1)


@jax.jit
def kernel(hidden, weight):
    n, d = hidden.shape
    grid = (n // BLK,)
    idx, wgt = pl.pallas_call(
        _gate_block,
        grid=grid,
        in_specs=[
            pl.BlockSpec((BLK, d), lambda i: (i, 0)),
            pl.BlockSpec((E, d), lambda i: (0, 0)),
        ],
        out_specs=[
            pl.BlockSpec((BLK, K), lambda i: (i, 0)),
            pl.BlockSpec((BLK, K), lambda i: (i, 0)),
        ],
        out_shape=[
            jax.ShapeDtypeStruct((n, K), jnp.int32),
            jax.ShapeDtypeStruct((n, K), jnp.float32),
        ],
    )(hidden, weight)
    return idx, wgt


# speedup vs baseline: 1.0164x; 1.0164x over previous
import jax
import jax.numpy as jnp
from jax.experimental import pallas as pl
from jax.experimental.pallas import tpu as pltpu

E = 8
K = 2
BLK = 2048


def _gate_block(hid_ref, w_ref, idx_ref, wgt_ref):
    h = hid_ref[...]
    w = w_ref[...]
    logits = jax.lax.dot_general(
        h, w, (((1,), (1,)), ((), ())),
        preferred_element_type=jnp.float32)
    m = jnp.max(logits, axis=-1, keepdims=True)
    s = jnp.exp(logits - m)
    z = jnp.sum(s, axis=-1, keepdims=True)
    scores = s / z

    lane = jax.lax.broadcasted_iota(jnp.int32, scores.shape, 1)
    w1 = jnp.max(scores, axis=-1, keepdims=True)
    i1 = jnp.min(jnp.where(scores >= w1, lane, E), axis=-1, keepdims=True)
    masked = jnp.where(lane == i1, -jnp.inf, scores)
    w2 = jnp.max(masked, axis=-1, keepdims=True)
    i2 = jnp.min(jnp.where(masked >= w2, lane, E), axis=-1, keepdims=True)

    denom = w1 + w2 + 1e-20
    idx_ref[...] = jnp.concatenate([i1, i2], axis=-1)
    wgt_ref[...] = jnp.concatenate([w1 / denom, w2 / denom], axis=-1)


@jax.jit
def kernel(hidden, weight):
    n, d = hidden.shape
    grid = (n // BLK,)
    idx, wgt = pl.pallas_call(
        _gate_block,
        grid=grid,
        in_specs=[
            pl.BlockSpec((BLK, d), lambda i: (i, 0)),
            pl.BlockSpec((E, d), lambda i: (0, 0)),
        ],
        out_specs=[
            pl.BlockSpec((BLK, K), lambda i: (i, 0)),
            pl.BlockSpec((BLK, K), lambda i: (i, 0)),
        ],
        out_shape=[
            jax.ShapeDtypeStruct((n, K), jnp.int32),
            jax.ShapeDtypeStruct((n, K), jnp.float32),
        ],
        compiler_params=pltpu.CompilerParams(
            dimension_semantics=("parallel",)),
    )(hidden, weight)
    return idx, wgt


# speedup vs baseline: 1.1542x; 1.1356x over previous
import jax
import jax.numpy as jnp
from jax.experimental import pallas as pl
from jax.experimental.pallas import tpu as pltpu

E = 8
K = 2
BLK = 2048


def _gate_block(hid_ref, w_ref, idx_ref, wgt_ref):
    h = hid_ref[...]
    w = w_ref[...]
    logits = jax.lax.dot_general(
        w, h, (((1,), (1,)), ((), ())),
        preferred_element_type=jnp.float32)
    m = jnp.max(logits, axis=0, keepdims=True)
    s = jnp.exp(logits - m)
    z = jnp.sum(s, axis=0, keepdims=True)
    scores = s / z

    sub = jax.lax.broadcasted_iota(jnp.int32, scores.shape, 0)
    w1 = jnp.max(scores, axis=0, keepdims=True)
    i1 = jnp.min(jnp.where(scores >= w1, sub, E), axis=0, keepdims=True)
    masked = jnp.where(sub == i1, -jnp.inf, scores)
    w2 = jnp.max(masked, axis=0, keepdims=True)
    i2 = jnp.min(jnp.where(masked >= w2, sub, E), axis=0, keepdims=True)

    denom = w1 + w2 + 1e-20
    idx_ref[...] = jnp.concatenate([i1, i2], axis=0).T
    wgt_ref[...] = jnp.concatenate([w1 / denom, w2 / denom], axis=0).T


@jax.jit
def kernel(hidden, weight):
    n, d = hidden.shape
    grid = (n // BLK,)
    idx, wgt = pl.pallas_call(
        _gate_block,
        grid=grid,
        in_specs=[
            pl.BlockSpec((BLK, d), lambda i: (i, 0)),
            pl.BlockSpec((E, d), lambda i: (0, 0)),
        ],
        out_specs=[
            pl.BlockSpec((BLK, K), lambda i: (i, 0)),
            pl.BlockSpec((BLK, K), lambda i: (i, 0)),
        ],
        out_shape=[
            jax.ShapeDtypeStruct((n, K), jnp.int32),
            jax.ShapeDtypeStruct((n, K), jnp.float32),
        ],
        compiler_params=pltpu.CompilerParams(
            dimension_semantics=("parallel",)),
    )(hidden, weight)
    return idx, wgt


# speedup vs baseline: 1.1907x; 1.0316x over previous
import jax
import jax.numpy as jnp
from jax.experimental import pallas as pl
from jax.experimental.pallas import tpu as pltpu

E = 8
K = 2
BLK = 4096


def _gate_block(hid_ref, w_ref, idx_ref, wgt_ref):
    h = hid_ref[...]
    w = w_ref[...]
    logits = jax.lax.dot_general(
        w, h, (((1,), (1,)), ((), ())),
        preferred_element_type=jnp.float32)
    m = jnp.max(logits, axis=0, keepdims=True)
    s = jnp.exp(logits - m)
    z = jnp.sum(s, axis=0, keepdims=True)
    scores = s / z

    sub = jax.lax.broadcasted_iota(jnp.int32, scores.shape, 0)
    w1 = jnp.max(scores, axis=0, keepdims=True)
    i1 = jnp.min(jnp.where(scores >= w1, sub, E), axis=0, keepdims=True)
    masked = jnp.where(sub == i1, -jnp.inf, scores)
    w2 = jnp.max(masked, axis=0, keepdims=True)
    i2 = jnp.min(jnp.where(masked >= w2, sub, E), axis=0, keepdims=True)

    denom = w1 + w2 + 1e-20
    idx_ref[...] = jnp.concatenate([i1, i2], axis=0).T
    wgt_ref[...] = jnp.concatenate([w1 / denom, w2 / denom], axis=0).T


@jax.jit
def kernel(hidden, weight):
    n, d = hidden.shape
    grid = (n // BLK,)
    idx, wgt = pl.pallas_call(
        _gate_block,
        grid=grid,
        in_specs=[
            pl.BlockSpec((BLK, d), lambda i: (i, 0)),
            pl.BlockSpec((E, d), lambda i: (0, 0)),
        ],
        out_specs=[
            pl.BlockSpec((BLK, K), lambda i: (i, 0)),
            pl.BlockSpec((BLK, K), lambda i: (i, 0)),
        ],
        out_shape=[
            jax.ShapeDtypeStruct((n, K), jnp.int32),
            jax.ShapeDtypeStruct((n, K), jnp.float32),
        ],
        compiler_params=pltpu.CompilerParams(
            dimension_semantics=("parallel",)),
    )(hidden, weight)
    return idx, wgt
